# baseline (speedup 1.0000x reference)
import jax
import jax.numpy as jnp
from jax.experimental import pallas as pl
from jax.experimental.pallas import tpu as pltpu


def _partials_kernel(u_ref, p_ref, n_ref, out_ref):
    u = u_ref[...]
    p = p_ref[...]
    n = n_ref[...]

    a = jnp.sum(u * p, axis=1, keepdims=True)
    b = jnp.sum(u * n, axis=1, keepdims=True)
    sq = jnp.sum(u * u + p * p + n * n)

    sp = 1.0 / (1.0 + jnp.exp(-a))
    sn = 1.0 / (1.0 + jnp.exp(-b))
    wmf = jnp.sum(2.0 * (sp - 1.0) ** 2 + sn * sn)

    lane = jax.lax.broadcasted_iota(jnp.int32, (1, 8, 128), 2)
    sub = jax.lax.broadcasted_iota(jnp.int32, (1, 8, 128), 1)
    out_ref[...] = jnp.where((lane == 0) & (sub == 0), sq, 0.0) + \
                   jnp.where((lane == 1) & (sub == 0), wmf, 0.0)


def kernel(user_embedding, item_embedding, users, positive_items,
           negative_items, weight_decay):
    B = users.shape[0]
    D = user_embedding.shape[1]

    n_chunks = 4
    tile = 2048
    assert B % (n_chunks * 2 * tile) == 0
    bc = B // n_chunks
    tpc = bc // (2 * tile)

    vec_spec = pl.BlockSpec((tile, D), lambda c, t: (c * tpc + t, 0))
    call = pl.pallas_call(
        _partials_kernel,
        out_shape=jax.ShapeDtypeStruct((2 * tpc, 8, 128), jnp.float32),
        grid=(2, tpc),
        in_specs=[vec_spec, vec_spec, vec_spec],
        out_specs=pl.BlockSpec((1, 8, 128), lambda c, t: (c * tpc + t, 0, 0)),
        compiler_params=pltpu.CompilerParams(
            dimension_semantics=("parallel", "arbitrary")),
    )

    partials = []
    for h in range(n_chunks):
        s = h * bc
        u = user_embedding[jax.lax.dynamic_slice_in_dim(users, s, bc)]
        p = item_embedding[jax.lax.dynamic_slice_in_dim(positive_items, s, bc)]
        n = item_embedding[jax.lax.dynamic_slice_in_dim(negative_items, s, bc)]
        partials.append(call(u, p, n))

    allp = jnp.stack(partials)
    sq_total = jnp.sum(allp[:, :, 0, 0])
    wmf_total = jnp.sum(allp[:, :, 0, 1])
    return wmf_total / (2.0 * B) + weight_decay * 0.5 * sq_total / B
